# baseline (speedup 1.0000x reference)
import functools
import math

import jax
import jax.numpy as jnp
from jax.experimental import pallas as pl
from jax.experimental.pallas import tpu as pltpu

_LANES = 128


def _round_up(x, m):
    return ((x + m - 1) // m) * m


def _stem_kernel(pt_ref, wt_ref, bt_ref, o_ref, acc_ref, *, n_m, hw, tile_m, c_out):
    m = pl.program_id(1)

    @pl.when(m == 0)
    def _init():
        acc_ref[...] = jnp.zeros_like(acc_ref)

    y = jnp.dot(wt_ref[...], pt_ref[0], preferred_element_type=jnp.float32)
    y = y[0:c_out, :] + bt_ref[...]
    y = y * jax.nn.sigmoid(y)
    col = jax.lax.broadcasted_iota(jnp.int32, (c_out, tile_m), 1) + m * tile_m
    acc_ref[...] += jnp.where(col < hw, y, 0.0)

    @pl.when(m == n_m - 1)
    def _finish():
        o_ref[0] = jnp.sum(acc_ref[...], axis=1, keepdims=True) * (1.0 / hw)


def _erf_poly(x):
    a1, a2, a3, a4, a5 = 0.254829592, -0.284496736, 1.421413741, -1.453152027, 1.061405429
    p = 0.3275911
    s = jnp.where(x >= 0.0, 1.0, -1.0)
    z = jnp.abs(x)
    t = 1.0 / (1.0 + p * z)
    poly = t * (a1 + t * (a2 + t * (a3 + t * (a4 + t * a5))))
    return s * (1.0 - poly * jnp.exp(-z * z))


def _gelu(x):
    return 0.5 * x * (1.0 + _erf_poly(x * 0.7071067811865476))


def _head_kernel(x_ref, wa_ref, ba_ref, wb_ref, bb_ref, wc_ref, bc_ref, o_ref):
    h = jnp.dot(x_ref[...], wa_ref[...], preferred_element_type=jnp.float32) + ba_ref[...]
    h = _gelu(h)
    h = jnp.dot(h.astype(jnp.bfloat16), wb_ref[...],
                preferred_element_type=jnp.float32) + bb_ref[...]
    h = _gelu(h)
    o_ref[...] = jnp.dot(h.astype(jnp.bfloat16), wc_ref[...],
                         preferred_element_type=jnp.float32) + bc_ref[...]


def _im2col_t(x, kh, kw, stride, pad, m_pad):
    B, C, H, W = x.shape
    xb = x.astype(jnp.bfloat16)
    xp = jnp.pad(xb, ((0, 0), (0, 0), (pad, pad), (pad, pad)))
    Hp, Wp = H + 2 * pad, W + 2 * pad
    Ho = (Hp - kh) // stride + 1
    Wo = (Wp - kw) // stride + 1
    rows = []
    for i in range(kh):
        for j in range(kw):
            sl = jax.lax.slice(
                xp, (0, 0, i, j),
                (B, C, i + (Ho - 1) * stride + 1, j + (Wo - 1) * stride + 1),
                (1, 1, stride, stride))
            rows.append(sl.reshape(B, C, Ho * Wo))
    pt = jnp.concatenate(rows, axis=1)
    k = pt.shape[1]
    pt = jnp.pad(pt, ((0, 0), (0, _round_up(k, 32) - k), (0, m_pad - Ho * Wo)))
    return pt, Ho, Wo


def _stem_pool(x, w_stem, b_stem):
    B, C, H, W = x.shape
    hw_est = ((H + 1) // 2) * ((W + 1) // 2)
    tile_m = 512 if hw_est >= 512 else _round_up(hw_est, 128)
    m_pad = _round_up(hw_est, tile_m)
    pt, ho, wo = _im2col_t(x, 3, 3, stride=2, pad=1, m_pad=m_pad)
    hw = ho * wo
    n_m = m_pad // tile_m
    c_out = 48
    kdim = pt.shape[1]

    wt = jnp.pad(w_stem.T, ((0, 0), (0, kdim - w_stem.shape[0])))
    bt = b_stem[0, 0:c_out].reshape(c_out, 1)

    kern = functools.partial(_stem_kernel, n_m=n_m, hw=hw, tile_m=tile_m, c_out=c_out)
    out = pl.pallas_call(
        kern,
        out_shape=jax.ShapeDtypeStruct((B, c_out, 1), jnp.float32),
        grid=(B, n_m),
        in_specs=[
            pl.BlockSpec((1, kdim, tile_m), lambda b, m: (b, 0, m)),
            pl.BlockSpec((_LANES, kdim), lambda b, m: (0, 0)),
            pl.BlockSpec((c_out, 1), lambda b, m: (0, 0)),
        ],
        out_specs=pl.BlockSpec((1, c_out, 1), lambda b, m: (b, 0, 0)),
        scratch_shapes=[pltpu.VMEM((c_out, tile_m), jnp.float32)],
        compiler_params=pltpu.CompilerParams(
            dimension_semantics=("parallel", "arbitrary"),
            vmem_limit_bytes=32 * 1024 * 1024),
    )(pt, wt, bt)
    return out[:, :, 0]


def _head(pooled48, wa, ba, wb, bb, wc, bc):
    B = pooled48.shape[0]
    x48 = pooled48.astype(jnp.bfloat16)
    wa48 = wa[0:48, :]
    args = (x48, wa48, ba, wb, bb, wc, bc)
    spec = pl.BlockSpec(memory_space=pltpu.MemorySpace.VMEM)
    out = pl.pallas_call(
        _head_kernel,
        out_shape=jax.ShapeDtypeStruct((B, _LANES), jnp.float32),
        in_specs=[spec] * len(args),
        out_specs=spec,
        compiler_params=pltpu.CompilerParams(vmem_limit_bytes=32 * 1024 * 1024),
    )(*args)
    return out


@jax.jit
def _forward(x, w_stem, b_stem, wa, ba, wb, bb, wc, bc):
    pooled = _stem_pool(x, w_stem, b_stem)
    return _head(pooled, wa, ba, wb, bb, wc, bc)[:, :8]


def kernel(x, w_stem, b_stem, wa, ba, wb, bb, wc, bc):
    return _forward(x, w_stem, b_stem, wa, ba, wb, bb, wc, bc)


# speedup vs baseline: 7.9582x; 7.9582x over previous
import functools
import math

import jax
import jax.numpy as jnp
from jax.experimental import pallas as pl
from jax.experimental.pallas import tpu as pltpu

_LANES = 128
_CH = 8


def _round_up(x, m):
    return ((x + m - 1) // m) * m


def _stem_kernel(x_ref, s_ref, wt_ref, bt_ref, o_ref, q_ref, rhs_ref, acc_ref,
                 *, C, H, W, Ho, Wo, c_out):
    xb = x_ref[0].astype(jnp.bfloat16).reshape(C * H, W)
    q = jnp.dot(xb, s_ref[...], preferred_element_type=jnp.float32)
    q_ref[...] = q.astype(jnp.bfloat16)

    acc_ref[...] = jnp.zeros_like(acc_ref)
    rhs_ref[...] = jnp.zeros_like(rhs_ref)

    def chunk_body(ch, first):
        for c in range(C):
            if first:
                qc = q_ref[c * H:c * H + 32, :]
                off = 0
            else:
                base = pl.multiple_of(16 * ch - 16, 16)
                qc = q_ref[pl.ds(base + c * H, 32), :]
                off = 16
            for oh_l in range(_CH):
                for di in range(3):
                    if first and oh_l == 0 and di == 0:
                        continue
                    rel = off + 2 * oh_l + di - 1
                    row = qc[rel:rel + 1, :]
                    dst = oh_l * _LANES
                    for dj in range(3):
                        k = di * 9 + dj * 3 + c
                        rhs_ref[k:k + 1, dst:dst + _LANES] = \
                            row[:, dj * _LANES:(dj + 1) * _LANES]
        y = jnp.dot(wt_ref[...], rhs_ref[...], preferred_element_type=jnp.float32)
        y = y[0:c_out, :] + bt_ref[...]
        acc_ref[...] += y * jax.nn.sigmoid(y)
        return ch + 1

    chunk_body(0, True)
    n_chunks = Ho // _CH
    jax.lax.fori_loop(1, n_chunks, lambda ch, _: chunk_body(ch, False) * 0, 0)

    lane = jax.lax.broadcasted_iota(jnp.int32, (c_out, _CH * _LANES), 1) % _LANES
    pooled = jnp.sum(jnp.where(lane < Wo, acc_ref[...], 0.0),
                     axis=1, keepdims=True) * (1.0 / (Ho * Wo))
    o_ref[0] = pooled


def _erf_poly(x):
    a1, a2, a3, a4, a5 = 0.254829592, -0.284496736, 1.421413741, -1.453152027, 1.061405429
    p = 0.3275911
    s = jnp.where(x >= 0.0, 1.0, -1.0)
    z = jnp.abs(x)
    t = 1.0 / (1.0 + p * z)
    poly = t * (a1 + t * (a2 + t * (a3 + t * (a4 + t * a5))))
    return s * (1.0 - poly * jnp.exp(-z * z))


def _gelu(x):
    return 0.5 * x * (1.0 + _erf_poly(x * 0.7071067811865476))


def _head_kernel(x_ref, wa_ref, ba_ref, wb_ref, bb_ref, wc_ref, bc_ref, o_ref):
    h = jnp.dot(x_ref[...], wa_ref[...], preferred_element_type=jnp.float32) + ba_ref[...]
    h = _gelu(h)
    h = jnp.dot(h.astype(jnp.bfloat16), wb_ref[...],
                preferred_element_type=jnp.float32) + bb_ref[...]
    h = _gelu(h)
    o_ref[...] = jnp.dot(h.astype(jnp.bfloat16), wc_ref[...],
                         preferred_element_type=jnp.float32) + bc_ref[...]


def _col_select(W, Wo):
    j = jnp.arange(W)[:, None]
    col = jnp.arange(3 * _LANES)[None, :]
    ow = col % _LANES
    dj = col // _LANES
    sel = (ow < Wo) & (j == 2 * ow + dj - 1)
    return sel.astype(jnp.bfloat16)


def _stem_pool(x, w_stem, b_stem):
    B, C, H, W = x.shape
    Ho, Wo = (H + 1) // 2, (W + 1) // 2
    c_out = 48
    kdim = _round_up(3 * 3 * C, 32)

    s = _col_select(W, Wo)
    wt = jnp.pad(w_stem.T, ((0, 0), (0, kdim - w_stem.shape[0])))
    bt = b_stem[0, 0:c_out].reshape(c_out, 1)

    kern = functools.partial(_stem_kernel, C=C, H=H, W=W, Ho=Ho, Wo=Wo, c_out=c_out)
    out = pl.pallas_call(
        kern,
        out_shape=jax.ShapeDtypeStruct((B, c_out, 1), jnp.float32),
        grid=(B,),
        in_specs=[
            pl.BlockSpec((1, C, H, W), lambda b: (b, 0, 0, 0)),
            pl.BlockSpec((W, 3 * _LANES), lambda b: (0, 0)),
            pl.BlockSpec((_LANES, kdim), lambda b: (0, 0)),
            pl.BlockSpec((c_out, 1), lambda b: (0, 0)),
        ],
        out_specs=pl.BlockSpec((1, c_out, 1), lambda b: (b, 0, 0)),
        scratch_shapes=[
            pltpu.VMEM((C * H, 3 * _LANES), jnp.bfloat16),
            pltpu.VMEM((kdim, _CH * _LANES), jnp.bfloat16),
            pltpu.VMEM((c_out, _CH * _LANES), jnp.float32),
        ],
        compiler_params=pltpu.CompilerParams(
            dimension_semantics=("parallel",),
            vmem_limit_bytes=32 * 1024 * 1024),
    )(x, s, wt, bt)
    return out[:, :, 0]


def _head(pooled48, wa, ba, wb, bb, wc, bc):
    B = pooled48.shape[0]
    x48 = pooled48.astype(jnp.bfloat16)
    wa48 = wa[0:48, :]
    args = (x48, wa48, ba, wb, bb, wc, bc)
    spec = pl.BlockSpec(memory_space=pltpu.MemorySpace.VMEM)
    out = pl.pallas_call(
        _head_kernel,
        out_shape=jax.ShapeDtypeStruct((B, _LANES), jnp.float32),
        in_specs=[spec] * len(args),
        out_specs=spec,
        compiler_params=pltpu.CompilerParams(vmem_limit_bytes=32 * 1024 * 1024),
    )(*args)
    return out


@jax.jit
def _forward(x, w_stem, b_stem, wa, ba, wb, bb, wc, bc):
    pooled = _stem_pool(x, w_stem, b_stem)
    return _head(pooled, wa, ba, wb, bb, wc, bc)[:, :8]


def kernel(x, w_stem, b_stem, wa, ba, wb, bb, wc, bc):
    return _forward(x, w_stem, b_stem, wa, ba, wb, bb, wc, bc)


# speedup vs baseline: 10.0163x; 1.2586x over previous
import functools
import math

import jax
import jax.numpy as jnp
from jax.experimental import pallas as pl
from jax.experimental.pallas import tpu as pltpu

_LANES = 128
_CH = 8


def _round_up(x, m):
    return ((x + m - 1) // m) * m


def _stem_kernel(x_ref, s_ref, wt_ref, bt_ref, o_ref, q_ref, rhs_a, rhs_b, acc_ref,
                 *, C, H, W, Ho, Wo, c_out):
    xb = x_ref[0].astype(jnp.bfloat16).reshape(C * H, W)
    q = jnp.dot(xb, s_ref[...], preferred_element_type=jnp.float32)
    q_ref[...] = q.astype(jnp.bfloat16)

    acc_ref[...] = jnp.zeros_like(acc_ref)
    rhs_a[...] = jnp.zeros_like(rhs_a)
    rhs_b[...] = jnp.zeros_like(rhs_b)

    def assemble(ch_base, rhs_ref, first):
        off = 0 if first else 16
        for c in range(C):
            if first:
                qc = q_ref[c * H:c * H + 32, :]
            else:
                qc = q_ref[pl.ds(ch_base + c * H, 32), :]
            for t in range(-1, 16):
                if first and t < 0:
                    continue
                row = qc[off + t:off + t + 1, :]
                for di in range(3):
                    num = t + 1 - di
                    if num % 2 or not 0 <= num // 2 < _CH:
                        continue
                    dst = (num // 2) * _LANES
                    for dj in range(3):
                        k = di * 9 + dj * 3 + c
                        rhs_ref[k:k + 1, dst:dst + _LANES] = \
                            row[:, dj * _LANES:(dj + 1) * _LANES]

    def dotacc(rhs_ref):
        y = jnp.dot(wt_ref[...], rhs_ref[...], preferred_element_type=jnp.float32)
        y = y[0:c_out, :] + bt_ref[...]
        acc_ref[...] += y * jax.nn.sigmoid(y)

    n_chunks = Ho // _CH
    assemble(0, rhs_a, True)

    def body2(i, _):
        b1 = pl.multiple_of(32 * i, 16)
        b2 = pl.multiple_of(32 * i + 16, 16)
        assemble(b1, rhs_b, False)
        dotacc(rhs_a)
        assemble(b2, rhs_a, False)
        dotacc(rhs_b)
        return 0

    jax.lax.fori_loop(0, (n_chunks - 2) // 2, body2, 0)
    assemble(16 * (n_chunks - 1) - 16, rhs_b, False)
    dotacc(rhs_a)
    dotacc(rhs_b)

    lane = jax.lax.broadcasted_iota(jnp.int32, (c_out, _CH * _LANES), 1) % _LANES
    pooled = jnp.sum(jnp.where(lane < Wo, acc_ref[...], 0.0),
                     axis=1, keepdims=True) * (1.0 / (Ho * Wo))
    o_ref[0] = pooled


def _erf_poly(x):
    a1, a2, a3, a4, a5 = 0.254829592, -0.284496736, 1.421413741, -1.453152027, 1.061405429
    p = 0.3275911
    s = jnp.where(x >= 0.0, 1.0, -1.0)
    z = jnp.abs(x)
    t = 1.0 / (1.0 + p * z)
    poly = t * (a1 + t * (a2 + t * (a3 + t * (a4 + t * a5))))
    return s * (1.0 - poly * jnp.exp(-z * z))


def _gelu(x):
    return 0.5 * x * (1.0 + _erf_poly(x * 0.7071067811865476))


def _head_kernel(x_ref, wa_ref, ba_ref, wb_ref, bb_ref, wc_ref, bc_ref, o_ref):
    h = jnp.dot(x_ref[...], wa_ref[...], preferred_element_type=jnp.float32) + ba_ref[...]
    h = _gelu(h)
    h = jnp.dot(h.astype(jnp.bfloat16), wb_ref[...],
                preferred_element_type=jnp.float32) + bb_ref[...]
    h = _gelu(h)
    o_ref[...] = jnp.dot(h.astype(jnp.bfloat16), wc_ref[...],
                         preferred_element_type=jnp.float32) + bc_ref[...]


def _col_select(W, Wo):
    j = jnp.arange(W)[:, None]
    col = jnp.arange(3 * _LANES)[None, :]
    ow = col % _LANES
    dj = col // _LANES
    sel = (ow < Wo) & (j == 2 * ow + dj - 1)
    return sel.astype(jnp.bfloat16)


def _stem_pool(x, w_stem, b_stem):
    B, C, H, W = x.shape
    Ho, Wo = (H + 1) // 2, (W + 1) // 2
    c_out = 48
    kdim = _round_up(3 * 3 * C, 32)

    s = _col_select(W, Wo)
    wt = jnp.pad(w_stem.T, ((0, 0), (0, kdim - w_stem.shape[0])))
    bt = b_stem[0, 0:c_out].reshape(c_out, 1)

    kern = functools.partial(_stem_kernel, C=C, H=H, W=W, Ho=Ho, Wo=Wo, c_out=c_out)
    out = pl.pallas_call(
        kern,
        out_shape=jax.ShapeDtypeStruct((B, c_out, 1), jnp.float32),
        grid=(B,),
        in_specs=[
            pl.BlockSpec((1, C, H, W), lambda b: (b, 0, 0, 0)),
            pl.BlockSpec((W, 3 * _LANES), lambda b: (0, 0)),
            pl.BlockSpec((_LANES, kdim), lambda b: (0, 0)),
            pl.BlockSpec((c_out, 1), lambda b: (0, 0)),
        ],
        out_specs=pl.BlockSpec((1, c_out, 1), lambda b: (b, 0, 0)),
        scratch_shapes=[
            pltpu.VMEM((C * H, 3 * _LANES), jnp.bfloat16),
            pltpu.VMEM((kdim, _CH * _LANES), jnp.bfloat16),
            pltpu.VMEM((kdim, _CH * _LANES), jnp.bfloat16),
            pltpu.VMEM((c_out, _CH * _LANES), jnp.float32),
        ],
        compiler_params=pltpu.CompilerParams(
            dimension_semantics=("parallel",),
            vmem_limit_bytes=32 * 1024 * 1024),
    )(x, s, wt, bt)
    return out[:, :, 0]


def _head(pooled48, wa, ba, wb, bb, wc, bc):
    B = pooled48.shape[0]
    x48 = pooled48.astype(jnp.bfloat16)
    wa48 = wa[0:48, :]
    args = (x48, wa48, ba, wb, bb, wc, bc)
    spec = pl.BlockSpec(memory_space=pltpu.MemorySpace.VMEM)
    out = pl.pallas_call(
        _head_kernel,
        out_shape=jax.ShapeDtypeStruct((B, _LANES), jnp.float32),
        in_specs=[spec] * len(args),
        out_specs=spec,
        compiler_params=pltpu.CompilerParams(vmem_limit_bytes=32 * 1024 * 1024),
    )(*args)
    return out


@jax.jit
def _forward(x, w_stem, b_stem, wa, ba, wb, bb, wc, bc):
    pooled = _stem_pool(x, w_stem, b_stem)
    return _head(pooled, wa, ba, wb, bb, wc, bc)[:, :8]


def kernel(x, w_stem, b_stem, wa, ba, wb, bb, wc, bc):
    return _forward(x, w_stem, b_stem, wa, ba, wb, bb, wc, bc)
